# baseline (speedup 1.0000x reference)
import functools
import jax
import jax.numpy as jnp
from jax.experimental import pallas as pl
from jax.experimental.pallas import tpu as pltpu
from jax.experimental.pallas import tpu_sc as plsc

_N = 10000
_D = 16
_F = 256
_NSC = 3200
_SC_NB = 4
_SC_TILES = 32
_TC_BN = 200
_LANES = 16


def _tc_body(a1_ref, a2_ref, ft_ref, out_ref):
    a = a1_ref[...][None, :, :] + a2_ref[...]
    a = jnp.maximum(a, 0.01 * a)
    m = jnp.max(a, axis=0, keepdims=True)
    e = jnp.exp(a - m)
    s = jnp.sum(e, axis=0)
    w = jnp.sum(e * ft_ref[...], axis=0)
    out_ref[...] = w / s


def _tc_part(a1, a2, ft):
    n, f = a1.shape
    d = a2.shape[0]
    bn = _TC_BN
    off = _NSC // bn
    grid = ((n - _NSC) // bn,)
    return pl.pallas_call(
        _tc_body,
        grid=grid,
        in_specs=[
            pl.BlockSpec((bn, f), lambda i: (i + off, 0)),
            pl.BlockSpec((d, bn, f), lambda i: (0, i + off, 0)),
            pl.BlockSpec((d, bn, f), lambda i: (0, i + off, 0)),
        ],
        out_specs=pl.BlockSpec((bn, f), lambda i: (i + off, 0)),
        out_shape=jax.ShapeDtypeStruct((n, f), a1.dtype),
    )(a1, a2, ft)


def _sc_compute_block(a1_v, a2_v, ft_v, out_v):
    nb = a1_v.shape[0]

    def _tree(vals, op):
        while len(vals) > 1:
            vals = [op(vals[i], vals[i + 1]) for i in range(0, len(vals) - 1, 2)] + (
                [vals[-1]] if len(vals) % 2 else []
            )
        return vals[0]

    @pl.loop(0, nb)
    def _node(nn):
        @pl.loop(0, _F, step=_LANES, unroll=2)
        def _fvec(f0):
            sl = pl.ds(f0, _LANES)
            va1 = a1_v[nn, sl]
            xs = []
            for d in range(_D):
                x = a2_v[d, nn, sl] + va1
                xs.append(jnp.maximum(x, x * 0.01))
            m = _tree(list(xs), jnp.maximum)
            es = [jnp.exp(x - m) for x in xs]
            ws = [e * ft_v[d, nn, sl] for d, e in enumerate(es)]
            s = _tree(es, jnp.add)
            acc = _tree(ws, jnp.add)
            out_v[nn, sl] = acc / s


def _sc_part(a1, a2, ft):
    mesh = plsc.VectorSubcoreMesh(core_axis_name="c", subcore_axis_name="s")

    @functools.partial(
        pl.kernel,
        out_type=jax.ShapeDtypeStruct((_NSC, _F), jnp.float32),
        mesh=mesh,
    )
    def sc_kern(a1_hbm, a2_hbm, ft_hbm, out_hbm):
        steps = _NSC // _SC_NB
        per_tile = steps // _SC_TILES

        def _perm(i):
            return (i % _SC_TILES) * per_tile + i // _SC_TILES

        pltpu.emit_pipeline(
            _sc_compute_block,
            grid=(steps,),
            in_specs=[
                pl.BlockSpec((_SC_NB, _F), lambda i: (_perm(i), 0)),
                pl.BlockSpec((_D, _SC_NB, _F), lambda i: (0, _perm(i), 0)),
                pl.BlockSpec((_D, _SC_NB, _F), lambda i: (0, _perm(i), 0)),
            ],
            out_specs=[pl.BlockSpec((_SC_NB, _F), lambda i: (_perm(i), 0))],
            core_axis_name=("c", "s"),
            dimension_semantics=(pltpu.PARALLEL,),
        )(a1_hbm, a2_hbm, ft_hbm, out_hbm)

    return sc_kern(a1, a2, ft)


def kernel(a1, a2, ft):
    sc_out = _sc_part(a1, a2, ft)
    tc_out = _tc_part(a1, a2, ft)
    return jax.lax.dynamic_update_slice(tc_out, sc_out, (0, 0))


# speedup vs baseline: 1.0242x; 1.0242x over previous
import functools
import jax
import jax.numpy as jnp
from jax.experimental import pallas as pl
from jax.experimental.pallas import tpu as pltpu
from jax.experimental.pallas import tpu_sc as plsc

_N = 10000
_D = 16
_F = 256
_NSC = 3200
_SC_NB = 4
_SC_TILES = 32
_TC_BN = 200
_LANES = 16


def _tc_body(a1_ref, a2_ref, ft_ref, out_ref):
    a = a1_ref[...][None, :, :] + a2_ref[...]
    a = jnp.maximum(a, 0.01 * a)
    m = jnp.max(a, axis=0, keepdims=True)
    e = jnp.exp(a - m)
    s = jnp.sum(e, axis=0)
    w = jnp.sum(e * ft_ref[...], axis=0)
    out_ref[...] = w / s


def _tc_part(a1, a2, ft):
    n, f = a1.shape
    d = a2.shape[0]
    bn = _TC_BN
    off = _NSC // bn
    grid = ((n - _NSC) // bn,)
    return pl.pallas_call(
        _tc_body,
        grid=grid,
        in_specs=[
            pl.BlockSpec((bn, f), lambda i: (i + off, 0)),
            pl.BlockSpec((d, bn, f), lambda i: (0, i + off, 0)),
            pl.BlockSpec((d, bn, f), lambda i: (0, i + off, 0)),
        ],
        out_specs=pl.BlockSpec((bn, f), lambda i: (i + off, 0)),
        out_shape=jax.ShapeDtypeStruct((n, f), a1.dtype),
    )(a1, a2, ft)


def _sc_compute_block(a1_v, a2_v, ft_v, out_v):
    nb = a1_v.shape[0]

    def _tree(vals, op):
        while len(vals) > 1:
            vals = [op(vals[i], vals[i + 1]) for i in range(0, len(vals) - 1, 2)] + (
                [vals[-1]] if len(vals) % 2 else []
            )
        return vals[0]

    @pl.loop(0, nb)
    def _node(nn):
        @plsc.parallel_loop(0, _F, step=_LANES, unroll=2)
        def _fvec(f0):
            sl = pl.ds(f0, _LANES)
            va1 = a1_v[nn, sl]
            xs = []
            for d in range(_D):
                x = a2_v[d, nn, sl] + va1
                xs.append(jnp.maximum(x, x * 0.01))
            m = _tree(list(xs), jnp.maximum)
            es = [jnp.exp(x - m) for x in xs]
            ws = [e * ft_v[d, nn, sl] for d, e in enumerate(es)]
            s = _tree(es, jnp.add)
            acc = _tree(ws, jnp.add)
            out_v[nn, sl] = acc / s


def _sc_part(a1, a2, ft):
    mesh = plsc.VectorSubcoreMesh(core_axis_name="c", subcore_axis_name="s")

    @functools.partial(
        pl.kernel,
        out_type=jax.ShapeDtypeStruct((_NSC, _F), jnp.float32),
        mesh=mesh,
    )
    def sc_kern(a1_hbm, a2_hbm, ft_hbm, out_hbm):
        steps = _NSC // _SC_NB
        per_tile = steps // _SC_TILES

        def _perm(i):
            return (i % _SC_TILES) * per_tile + i // _SC_TILES

        pltpu.emit_pipeline(
            _sc_compute_block,
            grid=(steps,),
            in_specs=[
                pl.BlockSpec((_SC_NB, _F), lambda i: (_perm(i), 0)),
                pl.BlockSpec((_D, _SC_NB, _F), lambda i: (0, _perm(i), 0)),
                pl.BlockSpec((_D, _SC_NB, _F), lambda i: (0, _perm(i), 0)),
            ],
            out_specs=[pl.BlockSpec((_SC_NB, _F), lambda i: (_perm(i), 0))],
            core_axis_name=("c", "s"),
            dimension_semantics=(pltpu.PARALLEL,),
        )(a1_hbm, a2_hbm, ft_hbm, out_hbm)

    return sc_kern(a1, a2, ft)


def kernel(a1, a2, ft):
    sc_out = _sc_part(a1, a2, ft)
    tc_out = _tc_part(a1, a2, ft)
    return jax.lax.dynamic_update_slice(tc_out, sc_out, (0, 0))
